# baseline (speedup 1.0000x reference)
import functools

import jax
import jax.numpy as jnp
import numpy as np
from jax import lax
from jax.experimental import pallas as pl
from jax.experimental.pallas import tpu as pltpu
from jax.experimental.pallas import tpu_sc as plsc

_F = 26
_D = 16
_B = 16384
_NC, _NS = 2, 16
_NW = _NC * _NS
_E = _B * _F
_EPW = _E // _NW
_CHUNK = 128
_NCH = _EPW // _CHUNK

_OFFSETS = np.array([0] + list(np.cumsum([100000] * _F)[:-1]), dtype=np.int32)


def _sc_gather_body(idx_hbm, emb_hbm, fc_hbm, emb_out, fc_out,
                    idx_v, ebuf, fbuf, sem_e, sem_f):
    c = lax.axis_index("c")
    s = lax.axis_index("s")
    wid = s * _NC + c
    pltpu.sync_copy(idx_hbm.at[wid], idx_v)
    base = wid * _EPW

    def step(j, carry):
        ge = pltpu.async_copy(emb_hbm.at[idx_v.at[j]], ebuf, sem_e)
        gf = pltpu.async_copy(fc_hbm.at[idx_v.at[j]], fbuf, sem_f)
        ge.wait()
        gf.wait()
        off = base + j * _CHUNK
        pltpu.sync_copy(ebuf, emb_out.at[pl.ds(off, _CHUNK)])
        pltpu.sync_copy(fbuf, fc_out.at[pl.ds(off, _CHUNK)])
        return carry

    lax.fori_loop(0, _NCH, step, 0)


@functools.partial(jax.jit, static_argnums=())
def _sc_gather(idx, emb_table, fc_table):
    mesh = plsc.VectorSubcoreMesh(core_axis_name="c", subcore_axis_name="s",
                                  num_cores=_NC, num_subcores=_NS)
    k = pl.kernel(
        _sc_gather_body,
        out_type=(jax.ShapeDtypeStruct((_E, _D), jnp.float32),
                  jax.ShapeDtypeStruct((_E,), jnp.float32)),
        mesh=mesh,
        scratch_types=[
            pltpu.VMEM((_NCH, _CHUNK), jnp.int32),
            pltpu.VMEM((_CHUNK, _D), jnp.float32),
            pltpu.VMEM((_CHUNK,), jnp.float32),
            pltpu.SemaphoreType.DMA,
            pltpu.SemaphoreType.DMA,
        ],
        compiler_params=pltpu.CompilerParams(use_tc_tiling_on_sc=False),
    )
    return k(idx, emb_table, fc_table.reshape(-1))


def _tc_mlp_body(e_ref, fc_ref, w1_ref, b1_ref, w2_ref, b2_ref, w3_ref,
                 s_ref, bias_ref, o_ref):
    e = e_ref[...]
    f32 = jnp.float32
    h1 = jnp.maximum(
        jnp.dot(e, w1_ref[...], preferred_element_type=f32) + b1_ref[...], 0.0)
    h2 = jnp.maximum(
        jnp.dot(h1, w2_ref[...], preferred_element_type=f32) + b2_ref[...], 0.0)
    deep = jnp.dot(h2, w3_ref[...], preferred_element_type=f32)
    sf = jnp.dot(e, s_ref[...], preferred_element_type=f32)
    second = 0.5 * (jnp.sum(sf * sf, axis=1, keepdims=True)
                    - jnp.sum(e * e, axis=1, keepdims=True))
    first = jnp.sum(fc_ref[...], axis=1, keepdims=True)
    logit = first + second + deep + bias_ref[0, 0]
    o_ref[...] = jax.nn.sigmoid(logit)


def _tc_mlp(emb_flat, fc_mat, W1, b1, W2, b2, W3, smat, bias_tot):
    BB = 2048
    grid = (_B // BB,)
    return pl.pallas_call(
        _tc_mlp_body,
        grid=grid,
        in_specs=[
            pl.BlockSpec((BB, _F * _D), lambda i: (i, 0)),
            pl.BlockSpec((BB, _F), lambda i: (i, 0)),
            pl.BlockSpec((_F * _D, 128), lambda i: (0, 0)),
            pl.BlockSpec((1, 128), lambda i: (0, 0)),
            pl.BlockSpec((128, 64), lambda i: (0, 0)),
            pl.BlockSpec((1, 64), lambda i: (0, 0)),
            pl.BlockSpec((64, 1), lambda i: (0, 0)),
            pl.BlockSpec((_F * _D, _D), lambda i: (0, 0)),
            pl.BlockSpec((1, 1), lambda i: (0, 0)),
        ],
        out_specs=pl.BlockSpec((BB, 1), lambda i: (i, 0)),
        out_shape=jax.ShapeDtypeStruct((_B, 1), jnp.float32),
        compiler_params=pltpu.CompilerParams(
            dimension_semantics=("arbitrary",),
        ),
    )(emb_flat, fc_mat, W1, b1, W2, b2, W3, smat, bias_tot)


def kernel(x, emb_table, fc_table, bias, W1, b1, W2, b2, W3, b3):
    idx = (x.astype(jnp.int32) + jnp.asarray(_OFFSETS)[None, :])
    idx = idx.reshape(_NW, _NCH, _CHUNK)
    emb_rows, fc_rows = _sc_gather(idx, emb_table, fc_table)
    emb_flat = emb_rows.reshape(_B, _F * _D)
    fc_mat = fc_rows.reshape(_B, _F)
    smat = jnp.tile(jnp.eye(_D, dtype=jnp.float32), (_F, 1))
    bias_tot = (bias + b3).reshape(1, 1)
    out = _tc_mlp(emb_flat, fc_mat, W1, b1.reshape(1, -1), W2,
                  b2.reshape(1, -1), W3, smat, bias_tot)
    return out


# speedup vs baseline: 1.0579x; 1.0579x over previous
import functools

import jax
import jax.numpy as jnp
import numpy as np
from jax import lax
from jax.experimental import pallas as pl
from jax.experimental.pallas import tpu as pltpu
from jax.experimental.pallas import tpu_sc as plsc

_F = 26
_D = 16
_B = 16384
_NC, _NS = 2, 16
_NW = _NC * _NS
_E = _B * _F
_EPW = _E // _NW
_CHUNK = 128
_NCH = _EPW // _CHUNK

_OFFSETS = np.array([0] + list(np.cumsum([100000] * _F)[:-1]), dtype=np.int32)


_K = 8
_NG = _NCH // _K
_CPW = _EPW // _CHUNK


def _sc_gather_body(idx_hbm, emb_hbm, fc_hbm, emb_out, fc_out,
                    idx_v, ebuf, fbuf,
                    sge0, sgf0, swe0, swf0, sge1, sgf1, swe1, swf1):
    c = lax.axis_index("c")
    s = lax.axis_index("s")
    wid = s * _NC + c
    pltpu.sync_copy(idx_hbm.at[wid], idx_v)
    crow = wid * _CPW

    banks = (
        (ebuf.at[0], fbuf.at[0], sge0, sgf0, swe0, swf0),
        (ebuf.at[1], fbuf.at[1], sge1, sgf1, swe1, swf1),
    )

    def issue_gathers(tg, bank):
        eb, fb, sge, sgf = bank[0], bank[1], bank[2], bank[3]
        for b in range(_K):
            g = tg * _K + b
            pltpu.async_copy(emb_hbm.at[idx_v.at[g]], eb.at[b], sge)
            pltpu.async_copy(fc_hbm.at[idx_v.at[g]], fb.at[b], sgf)

    def wait_gathers(tg, bank):
        eb, fb, sge, sgf = bank[0], bank[1], bank[2], bank[3]
        for b in range(_K):
            g = tg * _K + b
            pltpu.make_async_copy(emb_hbm.at[idx_v.at[g]], eb.at[b], sge).wait()
            pltpu.make_async_copy(fc_hbm.at[idx_v.at[g]], fb.at[b], sgf).wait()

    def issue_writes(tg, bank):
        eb, fb, swe, swf = bank[0], bank[1], bank[4], bank[5]
        row = crow + tg * _K
        pltpu.async_copy(eb, emb_out.at[pl.ds(row, _K)], swe)
        pltpu.async_copy(fb, fc_out.at[pl.ds(row, _K)], swf)

    def wait_writes(tg, bank):
        eb, fb, swe, swf = bank[0], bank[1], bank[4], bank[5]
        row = crow + tg * _K
        pltpu.make_async_copy(eb, emb_out.at[pl.ds(row, _K)], swe).wait()
        pltpu.make_async_copy(fb, fc_out.at[pl.ds(row, _K)], swf).wait()

    issue_gathers(0, banks[0])

    def body(t, carry):
        def do(bank, obank):
            @pl.when(t + 1 < _NG)
            def _():
                @pl.when(t >= 1)
                def _():
                    wait_writes(t - 1, obank)
                issue_gathers(t + 1, obank)
            wait_gathers(t, bank)
            issue_writes(t, bank)

        @pl.when(t % 2 == 0)
        def _():
            do(banks[0], banks[1])

        @pl.when(t % 2 == 1)
        def _():
            do(banks[1], banks[0])

        return carry

    lax.fori_loop(0, _NG, body, 0)
    wait_writes(_NG - 2, banks[(_NG - 2) % 2])
    wait_writes(_NG - 1, banks[(_NG - 1) % 2])


@functools.partial(jax.jit, static_argnums=())
def _sc_gather(idx, emb_table, fc_table):
    mesh = plsc.VectorSubcoreMesh(core_axis_name="c", subcore_axis_name="s",
                                  num_cores=_NC, num_subcores=_NS)
    k = pl.kernel(
        _sc_gather_body,
        out_type=(jax.ShapeDtypeStruct((_E // _CHUNK, _CHUNK, _D), jnp.float32),
                  jax.ShapeDtypeStruct((_E // _CHUNK, _CHUNK), jnp.float32)),
        mesh=mesh,
        scratch_types=[
            pltpu.VMEM((_NCH, _CHUNK), jnp.int32),
            pltpu.VMEM((2, _K, _CHUNK, _D), jnp.float32),
            pltpu.VMEM((2, _K, _CHUNK), jnp.float32),
        ] + [pltpu.SemaphoreType.DMA] * 8,
        compiler_params=pltpu.CompilerParams(use_tc_tiling_on_sc=False),
    )
    return k(idx, emb_table, fc_table.reshape(-1))


def _tc_mlp_body(e_ref, fc_ref, w1_ref, b1_ref, w2_ref, b2_ref, w3_ref,
                 s_ref, bias_ref, o_ref):
    e = e_ref[...]
    f32 = jnp.float32
    h1 = jnp.maximum(
        jnp.dot(e, w1_ref[...], preferred_element_type=f32) + b1_ref[...], 0.0)
    h2 = jnp.maximum(
        jnp.dot(h1, w2_ref[...], preferred_element_type=f32) + b2_ref[...], 0.0)
    deep = jnp.dot(h2, w3_ref[...], preferred_element_type=f32)
    sf = jnp.dot(e, s_ref[...], preferred_element_type=f32)
    second = 0.5 * (jnp.sum(sf * sf, axis=1, keepdims=True)
                    - jnp.sum(e * e, axis=1, keepdims=True))
    first = jnp.sum(fc_ref[...], axis=1, keepdims=True)
    logit = first + second + deep + bias_ref[0, 0]
    o_ref[...] = jax.nn.sigmoid(logit)


def _tc_mlp(emb_flat, fc_mat, W1, b1, W2, b2, W3, smat, bias_tot):
    BB = 2048
    grid = (_B // BB,)
    return pl.pallas_call(
        _tc_mlp_body,
        grid=grid,
        in_specs=[
            pl.BlockSpec((BB, _F * _D), lambda i: (i, 0)),
            pl.BlockSpec((BB, _F), lambda i: (i, 0)),
            pl.BlockSpec((_F * _D, 128), lambda i: (0, 0)),
            pl.BlockSpec((1, 128), lambda i: (0, 0)),
            pl.BlockSpec((128, 64), lambda i: (0, 0)),
            pl.BlockSpec((1, 64), lambda i: (0, 0)),
            pl.BlockSpec((64, 1), lambda i: (0, 0)),
            pl.BlockSpec((_F * _D, _D), lambda i: (0, 0)),
            pl.BlockSpec((1, 1), lambda i: (0, 0)),
        ],
        out_specs=pl.BlockSpec((BB, 1), lambda i: (i, 0)),
        out_shape=jax.ShapeDtypeStruct((_B, 1), jnp.float32),
        compiler_params=pltpu.CompilerParams(
            dimension_semantics=("arbitrary",),
        ),
    )(emb_flat, fc_mat, W1, b1, W2, b2, W3, smat, bias_tot)


def kernel(x, emb_table, fc_table, bias, W1, b1, W2, b2, W3, b3):
    idx = (x.astype(jnp.int32) + jnp.asarray(_OFFSETS)[None, :])
    idx = idx.reshape(_NW, _NCH, _CHUNK)
    emb_rows, fc_rows = _sc_gather(idx, emb_table, fc_table)
    emb_flat = emb_rows.reshape(_B, _F * _D)
    fc_mat = fc_rows.reshape(_B, _F)
    smat = jnp.tile(jnp.eye(_D, dtype=jnp.float32), (_F, 1))
    bias_tot = (bias + b3).reshape(1, 1)
    out = _tc_mlp(emb_flat, fc_mat, W1, b1.reshape(1, -1), W2,
                  b2.reshape(1, -1), W3, smat, bias_tot)
    return out
